# baseline (speedup 1.0000x reference)
import functools

import jax
import jax.numpy as jnp
from jax import lax
from jax.experimental import pallas as pl
from jax.experimental.pallas import tpu as pltpu
from jax.experimental.pallas import tpu_sc as plsc

D = 64
NB = 16384
NS = 50
NW = 32
B_PER_W = NB // NW
NBBLK = NB // 128
BBLK_PER_W = NBBLK // NW
UBLK = 128
UB_PER_W = B_PER_W // UBLK
N_UNITS = UB_PER_W * NS
DEPTH = 4


def _sc_embedding_gather(idx_flat, weight):
    mesh = plsc.VectorSubcoreMesh(core_axis_name="c", subcore_axis_name="s")

    @functools.partial(
        pl.kernel,
        mesh=mesh,
        out_type=jax.ShapeDtypeStruct((NS, D // 8, NBBLK, 8, 128),
                                      jnp.float32),
        scratch_types=(
            [pltpu.VMEM((UBLK,), jnp.int32)] * DEPTH
            + [pltpu.VMEM((UBLK, D), jnp.float32)] * DEPTH
            + [pltpu.VMEM((D // 8, UBLK // 128, 8, 128), jnp.float32)]
            * DEPTH
            + [pltpu.SemaphoreType.DMA] * (3 * DEPTH)
        ),
        compiler_params=pltpu.CompilerParams(
            use_tc_tiling_on_sc=False, needs_layout_passes=False),
    )
    def k(table_hbm, idx_hbm, out_hbm, *bufs):
        gidx_l = bufs[0:DEPTH]
        rows_l = bufs[DEPTH:2 * DEPTH]
        rt_l = bufs[2 * DEPTH:3 * DEPTH]
        sem_i = bufs[3 * DEPTH:4 * DEPTH]
        sem_g = bufs[4 * DEPTH:5 * DEPTH]
        sem_s = bufs[5 * DEPTH:6 * DEPTH]
        wid = lax.axis_index("s") * 2 + lax.axis_index("c")

        iota = lax.iota(jnp.int32, 16)

        def unit_sb(u):
            return u % NS, u // NS

        def load_gidx(u, gidx, sem):
            s, bb = unit_sb(u)
            pltpu.async_copy(
                idx_hbm.at[pl.ds(s * NB + wid * B_PER_W + bb * UBLK, UBLK)],
                gidx, sem)

        def wait_gidx(u, gidx, sem):
            s, bb = unit_sb(u)
            pltpu.make_async_copy(
                idx_hbm.at[pl.ds(s * NB + wid * B_PER_W + bb * UBLK, UBLK)],
                gidx, sem).wait()

        def gather(gidx, rows, sem):
            pltpu.async_copy(table_hbm.at[gidx], rows, sem)

        def wait_gather(gidx, rows, sem):
            pltpu.make_async_copy(table_hbm.at[gidx], rows, sem).wait()

        bases = [iota + bg * 16 for bg in range(UBLK // 16)]

        def transpose(rows, rt):
            @plsc.parallel_loop(0, D, unroll=4)
            def _(d):
                col = jnp.full((16,), d, jnp.int32)
                dhi = d // 8
                dlo = d % 8
                for bg in range(UBLK // 16):
                    v = plsc.load_gather(rows, [bases[bg], col])
                    rt[dhi, bg // 8, dlo, pl.ds((bg % 8) * 16, 16)] = v

        def out_slice(u):
            s, bb = unit_sb(u)
            nblk = UBLK // 128
            return out_hbm.at[s, :, pl.ds(wid * BBLK_PER_W + bb * nblk,
                                          nblk)]

        def store(u, rt, sem):
            pltpu.async_copy(rt, out_slice(u), sem)

        def wait_store(u, rt, sem):
            pltpu.make_async_copy(rt, out_slice(u), sem).wait()

        for j in range(DEPTH):
            load_gidx(j, gidx_l[j], sem_i[j])
        for j in range(DEPTH):
            wait_gidx(j, gidx_l[j], sem_i[j])
            gather(gidx_l[j], rows_l[j], sem_g[j])

        def body(i, carry):
            for j in range(DEPTH):
                u = DEPTH * i + j
                wait_gather(gidx_l[j], rows_l[j], sem_g[j])

                @pl.when(u + DEPTH < N_UNITS)
                def _():
                    load_gidx(u + DEPTH, gidx_l[j], sem_i[j])

                @pl.when(i > 0)
                def _():
                    wait_store(u - DEPTH, rt_l[j], sem_s[j])

                transpose(rows_l[j], rt_l[j])
                store(u, rt_l[j], sem_s[j])

                @pl.when(u + DEPTH < N_UNITS)
                def _():
                    wait_gidx(u + DEPTH, gidx_l[j], sem_i[j])
                    gather(gidx_l[j], rows_l[j], sem_g[j])

            return carry

        lax.fori_loop(0, N_UNITS // DEPTH, body, 0)
        for j in range(DEPTH):
            wait_store(N_UNITS - DEPTH + j, rt_l[j], sem_s[j])

    return k(weight, idx_flat)


NV = 1000000
TBLK = 2048


def _tc_relayout_table(wt):
    grid = (NV + TBLK - 1) // TBLK

    def body(wt_ref, out_ref):
        x = wt_ref[...]
        y = jnp.transpose(x)
        out_ref[:, 0:D] = y
        out_ref[:, D:128] = jnp.zeros((TBLK, 128 - D), jnp.float32)

    return pl.pallas_call(
        body,
        grid=(grid,),
        in_specs=[pl.BlockSpec((D, TBLK), lambda i: (0, i))],
        out_specs=pl.BlockSpec((TBLK, 128), lambda i: (i, 0)),
        out_shape=jax.ShapeDtypeStruct((NV, 128), jnp.float32),
    )(wt)


def kernel(input, weight):
    idx_flat = input.T.reshape(-1).astype(jnp.int32) * 2
    w_pad = _tc_relayout_table(weight.T)
    w2m = jnp.reshape(w_pad, (2 * NV, D))
    out5 = _sc_embedding_gather(idx_flat, w2m)
    return jnp.transpose(out5, (2, 4, 0, 1, 3)).reshape(NB, NS, D)
